# baseline (speedup 1.0000x reference)
import functools

import jax
import jax.numpy as jnp
from jax import lax
from jax.experimental import pallas as pl
from jax.experimental.pallas import tpu as pltpu
from jax.experimental.pallas import tpu_sc as plsc

_EMB = 128
_EDIM = 16
_N = 10000
_E = 320000
_BN_EPS = 1e-5

_NW = 32
_CH = 64
_NCH = 159
_EPW = _NCH * _CH
_EPAD = _EPW * _NW
_NAGG = 10112
_RPT = _NAGG // 16
_NBUF = 3
_NI = _NCH // _NBUF


def _edge_mlp(ea_pad, We1l, be1l, We2l, be2l):
    tile = 2048
    grid = (_EPAD // tile,)

    def body(ea_ref, w1_ref, b1_ref, w2_ref, b2_ref, o_ref):
        t = jnp.dot(ea_ref[...].astype(jnp.bfloat16),
                    w1_ref[...].astype(jnp.bfloat16),
                    preferred_element_type=jnp.float32)
        t = jnp.maximum(t + b1_ref[...], 0.0)
        o_ref[...] = (
            jnp.dot(t.astype(jnp.bfloat16), w2_ref[...].astype(jnp.bfloat16),
                    preferred_element_type=jnp.float32) + b2_ref[...]
        )

    return pl.pallas_call(
        body,
        grid=grid,
        in_specs=[
            pl.BlockSpec((tile, _EDIM), lambda i: (i, 0)),
            pl.BlockSpec((_EDIM, _EMB), lambda i: (0, 0)),
            pl.BlockSpec((1, _EMB), lambda i: (0, 0)),
            pl.BlockSpec((_EMB, _EMB), lambda i: (0, 0)),
            pl.BlockSpec((1, _EMB), lambda i: (0, 0)),
        ],
        out_specs=pl.BlockSpec((tile, _EMB), lambda i: (i, 0)),
        out_shape=jax.ShapeDtypeStruct((_EPAD, _EMB), jnp.float32),
    )(ea_pad, We1l, be1l.reshape(1, _EMB), We2l, be2l.reshape(1, _EMB))


def _sc_msgpass(h, e, src3d, dst3d):
    mesh = plsc.VectorSubcoreMesh(core_axis_name="c", subcore_axis_name="s")

    @functools.partial(
        pl.kernel,
        mesh=mesh,
        out_type=jax.ShapeDtypeStruct((2, _NAGG, _EMB), jnp.float32),
        scratch_types=[pltpu.VMEM((1, _CH), jnp.int32) for _ in range(2 * _NBUF)]
        + [pltpu.VMEM((_CH, _EMB), jnp.float32) for _ in range(2 * _NBUF)]
        + [pltpu.VMEM_SHARED((_NAGG, _EMB), jnp.float32)]
        + [pltpu.SemaphoreType.DMA for _ in range(5 * _NBUF)],
    )
    def k(h_hbm, e_hbm, src_hbm, dst_hbm, out_hbm, *rest):
        src_v = rest[0:_NBUF]
        dst_v = rest[_NBUF:2 * _NBUF]
        e_v = rest[2 * _NBUF:3 * _NBUF]
        g_v = rest[3 * _NBUF:4 * _NBUF]
        agg_sh = rest[4 * _NBUF]
        sem_e = rest[4 * _NBUF + 1:4 * _NBUF + 1 + _NBUF]
        sem_g = rest[4 * _NBUF + 1 + _NBUF:4 * _NBUF + 1 + 2 * _NBUF]
        sem_s = rest[4 * _NBUF + 1 + 2 * _NBUF:4 * _NBUF + 1 + 3 * _NBUF]
        sem_si = rest[4 * _NBUF + 1 + 3 * _NBUF:4 * _NBUF + 1 + 4 * _NBUF]
        sem_di = rest[4 * _NBUF + 1 + 4 * _NBUF:]
        c = lax.axis_index("c")
        s = lax.axis_index("s")
        wid = s * 2 + c

        def zrow(r, _):
            for j in range(8):
                e_v[0][r, pl.ds(j * 16, 16)] = jnp.zeros((16,), jnp.float32)
            return 0

        lax.fori_loop(0, _CH, zrow, 0)
        row0 = s * _RPT
        sizes = ((0, 64), (64, 64), (128, 64), (192, 64), (256, 64),
                 (320, 64), (384, 64), (448, 64), (512, 64), (576, 56))
        for off, sz in sizes:
            pltpu.sync_copy(e_v[0].at[pl.ds(0, sz)], agg_sh.at[pl.ds(row0 + off, sz)])
        plsc.subcore_barrier()

        ebase = wid * _EPW

        def src_async(kk, b):
            pltpu.async_copy(src_hbm.at[wid, pl.ds(kk, 1)], src_v[b], sem_si[b])

        def wait_src_idx(b):
            pltpu.make_async_copy(
                src_hbm.at[wid, pl.ds(0, 1)], src_v[b], sem_si[b]
            ).wait()

        def dst_async(kk, b):
            pltpu.async_copy(dst_hbm.at[wid, pl.ds(kk, 1)], dst_v[b], sem_di[b])

        def wait_dst_idx(b):
            pltpu.make_async_copy(
                dst_hbm.at[wid, pl.ds(0, 1)], dst_v[b], sem_di[b]
            ).wait()

        def issue_eg(kk, b):
            pltpu.async_copy(
                e_hbm.at[pl.ds(ebase + kk * _CH, _CH)], e_v[b], sem_e[b]
            )
            pltpu.async_copy(h_hbm.at[src_v[b].at[0]], g_v[b], sem_g[b])

        def wait_in(b):
            pltpu.make_async_copy(
                e_hbm.at[pl.ds(ebase, _CH)], e_v[b], sem_e[b]
            ).wait()
            pltpu.make_async_copy(
                h_hbm.at[src_v[b].at[0]], g_v[b], sem_g[b]
            ).wait()

        def wait_scatter(b):
            pltpu.make_async_copy(
                e_v[b], agg_sh.at[dst_v[b].at[0]], sem_s[b]
            ).wait()

        for j in range(_NBUF):
            pltpu.sync_copy(src_hbm.at[wid, pl.ds(j, 1)], src_v[j])
            pltpu.sync_copy(dst_hbm.at[wid, pl.ds(j, 1)], dst_v[j])
        issue_eg(0, 0)
        issue_eg(1, 1)

        def phase(i, u):
            kk = i * _NBUF + u
            nb = (u + 2) % _NBUF

            def refill():
                wait_scatter(nb)
                dst_async(kk + 2, nb)
                wait_src_idx(nb)
                issue_eg(kk + 2, nb)

            if u == 0:
                @pl.when(i > 0)
                def _():
                    refill()

                @pl.when(i == 0)
                def _():
                    dst_async(2, 2)
                    issue_eg(2, 2)
            else:
                @pl.when(i < _NI - 1)
                def _():
                    refill()

            wait_in(u)

            @pl.when(kk + _NBUF < _NCH)
            def _():
                src_async(kk + _NBUF, u)

            def ew(rr, _2):
                for rd in range(2):
                    r = rr * 2 + rd
                    for j in range(8):
                        d = pl.ds(j * 16, 16)
                        e_v[u][r, d] = jnp.maximum(
                            e_v[u][r, d] + g_v[u][r, d], 0.0
                        )
                return 0

            lax.fori_loop(0, _CH // 2, ew, 0)
            if u < 2:
                @pl.when(i > 0)
                def _():
                    wait_dst_idx(u)
            else:
                wait_dst_idx(u)
            pltpu.async_copy(
                e_v[u], agg_sh.at[dst_v[u].at[0]], sem_s[u], add=True
            )

        def body(i, _):
            for u in range(_NBUF):
                phase(i, u)
            return 0

        lax.fori_loop(0, _NI, body, 0)
        for b in range(_NBUF):
            wait_scatter(b)
        plsc.subcore_barrier()

        for off, sz in sizes:
            pltpu.sync_copy(agg_sh.at[pl.ds(row0 + off, sz)], e_v[0].at[pl.ds(0, sz)])
            pltpu.sync_copy(
                e_v[0].at[pl.ds(0, sz)], out_hbm.at[c, pl.ds(row0 + off, sz)]
            )

    return k(h, e, src3d, dst3d)


def _node_mlp(h, agg2, scale, W1l, s1, t1, W2l, s2, t2, relu_out):
    tile = 1000
    grid = (_N // tile,)

    def body(sc_ref, h_ref, a_ref, w1_ref, s1_ref, t1_ref, w2_ref, s2_ref,
             t2_ref, o_ref):
        u = sc_ref[0, 0] * h_ref[...] + a_ref[0] + a_ref[1]
        z = jnp.dot(u.astype(jnp.bfloat16), w1_ref[...].astype(jnp.bfloat16),
                    preferred_element_type=jnp.float32)
        z = jnp.maximum(z * s1_ref[...] + t1_ref[...], 0.0)
        z = jnp.dot(z.astype(jnp.bfloat16), w2_ref[...].astype(jnp.bfloat16),
                    preferred_element_type=jnp.float32)
        z = z * s2_ref[...] + t2_ref[...]
        if relu_out:
            z = jnp.maximum(z, 0.0)
        o_ref[...] = z + h_ref[...]

    return pl.pallas_call(
        body,
        grid=grid,
        in_specs=[
            pl.BlockSpec((1, 1), lambda i: (0, 0), memory_space=pltpu.SMEM),
            pl.BlockSpec((tile, _EMB), lambda i: (i, 0)),
            pl.BlockSpec((2, tile, _EMB), lambda i: (0, i, 0)),
            pl.BlockSpec((_EMB, 2 * _EMB), lambda i: (0, 0)),
            pl.BlockSpec((1, 2 * _EMB), lambda i: (0, 0)),
            pl.BlockSpec((1, 2 * _EMB), lambda i: (0, 0)),
            pl.BlockSpec((2 * _EMB, _EMB), lambda i: (0, 0)),
            pl.BlockSpec((1, _EMB), lambda i: (0, 0)),
            pl.BlockSpec((1, _EMB), lambda i: (0, 0)),
        ],
        out_specs=pl.BlockSpec((tile, _EMB), lambda i: (i, 0)),
        out_shape=jax.ShapeDtypeStruct((_N, _EMB), jnp.float32),
    )(
        scale.reshape(1, 1),
        h,
        agg2,
        W1l,
        s1.reshape(1, 2 * _EMB),
        t1.reshape(1, 2 * _EMB),
        W2l,
        s2.reshape(1, _EMB),
        t2.reshape(1, _EMB),
    )


def kernel(x, edge_index, edge_attr, batch, W1, b1, W2, b2, We1, be1, We2, be2,
           bn1_g, bn1_b, bn_g, bn_b, gin_eps):
    src = edge_index[0]
    dst = edge_index[1]
    npad = _EPAD - _E
    ar = jnp.arange(npad, dtype=jnp.int32)
    src_p = jnp.concatenate([src, (ar * 37) % _N]).reshape(_NW, _NCH, _CH)
    dst_p = jnp.concatenate([dst, _N + (ar % 112)]).reshape(_NW, _NCH, _CH)
    ea_p = jnp.concatenate(
        [edge_attr, jnp.zeros((npad, _EDIM), jnp.float32)], axis=0
    )

    inv = 1.0 / jnp.sqrt(1.0 + _BN_EPS)
    h = x
    L = W1.shape[0]
    for l in range(L):
        e = _edge_mlp(ea_p, We1[l], be1[l], We2[l], be2[l])
        parts = _sc_msgpass(h, e, src_p, dst_p)
        s1 = inv * bn1_g[l]
        t1 = b1[l] * s1 + bn1_b[l]
        s2 = inv * bn_g[l]
        t2 = b2[l] * s2 + bn_b[l]
        h = _node_mlp(h, parts[:, :_N], 1.0 + gin_eps[l], W1[l], s1, t1,
                      W2[l], s2, t2, relu_out=(l < L - 1))
    return h


# speedup vs baseline: 1.0960x; 1.0960x over previous
import functools

import jax
import jax.numpy as jnp
from jax import lax
from jax.experimental import pallas as pl
from jax.experimental.pallas import tpu as pltpu
from jax.experimental.pallas import tpu_sc as plsc

_EMB = 128
_EDIM = 16
_N = 10000
_E = 320000
_BN_EPS = 1e-5

_NW = 32
_CH = 64
_NCH = 159
_EPW = _NCH * _CH
_EPAD = _EPW * _NW
_NAGG = 10112
_RPT = _NAGG // 16
_NBUF = 3
_NI = _NCH // _NBUF


def _edge_mlp(ea, We1l, be1l, We2l, be2l):
    tile = 8192
    grid = (-(-_E // tile),)

    def body(ea_ref, w1_ref, b1_ref, w2_ref, b2_ref, o_ref):
        t = jnp.dot(ea_ref[...].astype(jnp.bfloat16),
                    w1_ref[...].astype(jnp.bfloat16),
                    preferred_element_type=jnp.float32)
        t = jnp.maximum(t + b1_ref[...], 0.0)
        o_ref[...] = (
            jnp.dot(t.astype(jnp.bfloat16), w2_ref[...].astype(jnp.bfloat16),
                    preferred_element_type=jnp.float32) + b2_ref[...]
        )

    return pl.pallas_call(
        body,
        grid=grid,
        in_specs=[
            pl.BlockSpec((tile, _EDIM), lambda i: (i, 0)),
            pl.BlockSpec((_EDIM, _EMB), lambda i: (0, 0)),
            pl.BlockSpec((1, _EMB), lambda i: (0, 0)),
            pl.BlockSpec((_EMB, _EMB), lambda i: (0, 0)),
            pl.BlockSpec((1, _EMB), lambda i: (0, 0)),
        ],
        out_specs=pl.BlockSpec((tile, _EMB), lambda i: (i, 0)),
        out_shape=jax.ShapeDtypeStruct((_EPAD, _EMB), jnp.float32),
    )(ea, We1l, be1l.reshape(1, _EMB), We2l, be2l.reshape(1, _EMB))


def _sc_msgpass(h, e, src3d, dst3d):
    mesh = plsc.VectorSubcoreMesh(core_axis_name="c", subcore_axis_name="s")

    @functools.partial(
        pl.kernel,
        mesh=mesh,
        out_type=jax.ShapeDtypeStruct((2, _NAGG, _EMB), jnp.float32),
        scratch_types=[pltpu.VMEM((1, _CH), jnp.int32) for _ in range(2 * _NBUF)]
        + [pltpu.VMEM((_CH, _EMB), jnp.float32) for _ in range(2 * _NBUF)]
        + [pltpu.VMEM_SHARED((_NAGG, _EMB), jnp.float32)]
        + [pltpu.SemaphoreType.DMA for _ in range(5 * _NBUF)],
    )
    def k(h_hbm, e_hbm, src_hbm, dst_hbm, out_hbm, *rest):
        src_v = rest[0:_NBUF]
        dst_v = rest[_NBUF:2 * _NBUF]
        e_v = rest[2 * _NBUF:3 * _NBUF]
        g_v = rest[3 * _NBUF:4 * _NBUF]
        agg_sh = rest[4 * _NBUF]
        sem_e = rest[4 * _NBUF + 1:4 * _NBUF + 1 + _NBUF]
        sem_g = rest[4 * _NBUF + 1 + _NBUF:4 * _NBUF + 1 + 2 * _NBUF]
        sem_s = rest[4 * _NBUF + 1 + 2 * _NBUF:4 * _NBUF + 1 + 3 * _NBUF]
        sem_si = rest[4 * _NBUF + 1 + 3 * _NBUF:4 * _NBUF + 1 + 4 * _NBUF]
        sem_di = rest[4 * _NBUF + 1 + 4 * _NBUF:]
        c = lax.axis_index("c")
        s = lax.axis_index("s")
        wid = s * 2 + c

        def zrow(r, _):
            for j in range(8):
                e_v[0][r, pl.ds(j * 16, 16)] = jnp.zeros((16,), jnp.float32)
            return 0

        lax.fori_loop(0, _CH, zrow, 0)
        row0 = s * _RPT
        sizes = ((0, 64), (64, 64), (128, 64), (192, 64), (256, 64),
                 (320, 64), (384, 64), (448, 64), (512, 64), (576, 56))
        for off, sz in sizes:
            pltpu.sync_copy(e_v[0].at[pl.ds(0, sz)], agg_sh.at[pl.ds(row0 + off, sz)])
        plsc.subcore_barrier()

        ebase = wid * _EPW

        def src_async(kk, b):
            pltpu.async_copy(src_hbm.at[wid, pl.ds(kk, 1)], src_v[b], sem_si[b])

        def wait_src_idx(b):
            pltpu.make_async_copy(
                src_hbm.at[wid, pl.ds(0, 1)], src_v[b], sem_si[b]
            ).wait()

        def dst_async(kk, b):
            pltpu.async_copy(dst_hbm.at[wid, pl.ds(kk, 1)], dst_v[b], sem_di[b])

        def wait_dst_idx(b):
            pltpu.make_async_copy(
                dst_hbm.at[wid, pl.ds(0, 1)], dst_v[b], sem_di[b]
            ).wait()

        def issue_eg(kk, b):
            pltpu.async_copy(
                e_hbm.at[pl.ds(ebase + kk * _CH, _CH)], e_v[b], sem_e[b]
            )
            pltpu.async_copy(h_hbm.at[src_v[b].at[0]], g_v[b], sem_g[b])

        def wait_in(b):
            pltpu.make_async_copy(
                e_hbm.at[pl.ds(ebase, _CH)], e_v[b], sem_e[b]
            ).wait()
            pltpu.make_async_copy(
                h_hbm.at[src_v[b].at[0]], g_v[b], sem_g[b]
            ).wait()

        def wait_scatter(b):
            pltpu.make_async_copy(
                e_v[b], agg_sh.at[dst_v[b].at[0]], sem_s[b]
            ).wait()

        for j in range(_NBUF):
            pltpu.sync_copy(src_hbm.at[wid, pl.ds(j, 1)], src_v[j])
            pltpu.sync_copy(dst_hbm.at[wid, pl.ds(j, 1)], dst_v[j])
        issue_eg(0, 0)
        issue_eg(1, 1)

        def phase(i, u):
            kk = i * _NBUF + u
            nb = (u + 2) % _NBUF

            def refill():
                wait_scatter(nb)
                dst_async(kk + 2, nb)
                wait_src_idx(nb)
                issue_eg(kk + 2, nb)

            if u == 0:
                @pl.when(i > 0)
                def _():
                    refill()

                @pl.when(i == 0)
                def _():
                    dst_async(2, 2)
                    issue_eg(2, 2)
            else:
                @pl.when(i < _NI - 1)
                def _():
                    refill()

            wait_in(u)

            @pl.when(kk + _NBUF < _NCH)
            def _():
                src_async(kk + _NBUF, u)

            def ew(rr, _2):
                for rd in range(2):
                    r = rr * 2 + rd
                    for j in range(8):
                        d = pl.ds(j * 16, 16)
                        e_v[u][r, d] = jnp.maximum(
                            e_v[u][r, d] + g_v[u][r, d], 0.0
                        )
                return 0

            lax.fori_loop(0, _CH // 2, ew, 0)
            if u < 2:
                @pl.when(i > 0)
                def _():
                    wait_dst_idx(u)
            else:
                wait_dst_idx(u)
            pltpu.async_copy(
                e_v[u], agg_sh.at[dst_v[u].at[0]], sem_s[u], add=True
            )

        def body(i, _):
            for u in range(_NBUF):
                phase(i, u)
            return 0

        lax.fori_loop(0, _NI, body, 0)
        for b in range(_NBUF):
            wait_scatter(b)
        plsc.subcore_barrier()

        for off, sz in sizes:
            pltpu.sync_copy(agg_sh.at[pl.ds(row0 + off, sz)], e_v[0].at[pl.ds(0, sz)])
            pltpu.sync_copy(
                e_v[0].at[pl.ds(0, sz)], out_hbm.at[c, pl.ds(row0 + off, sz)]
            )

    return k(h, e, src3d, dst3d)


def _node_mlp(h, agg2, scale, W1l, s1, t1, W2l, s2, t2, relu_out):
    ntile = 1000
    grid = (_N // ntile,)

    def body(sc_ref, h_ref, a_ref, w1_ref, s1_ref, t1_ref, w2_ref, s2_ref,
             t2_ref, o_ref):
        u = sc_ref[0, 0] * h_ref[...] + a_ref[0] + a_ref[1]
        z = jnp.dot(u.astype(jnp.bfloat16), w1_ref[...].astype(jnp.bfloat16),
                    preferred_element_type=jnp.float32)
        z = jnp.maximum(z * s1_ref[...] + t1_ref[...], 0.0)
        z = jnp.dot(z.astype(jnp.bfloat16), w2_ref[...].astype(jnp.bfloat16),
                    preferred_element_type=jnp.float32)
        z = z * s2_ref[...] + t2_ref[...]
        if relu_out:
            z = jnp.maximum(z, 0.0)
        o_ref[...] = z + h_ref[...]

    return pl.pallas_call(
        body,
        grid=grid,
        in_specs=[
            pl.BlockSpec((1, 1), lambda i: (0, 0), memory_space=pltpu.SMEM),
            pl.BlockSpec((ntile, _EMB), lambda i: (i, 0)),
            pl.BlockSpec((2, ntile, _EMB), lambda i: (0, i, 0)),
            pl.BlockSpec((_EMB, 2 * _EMB), lambda i: (0, 0)),
            pl.BlockSpec((1, 2 * _EMB), lambda i: (0, 0)),
            pl.BlockSpec((1, 2 * _EMB), lambda i: (0, 0)),
            pl.BlockSpec((2 * _EMB, _EMB), lambda i: (0, 0)),
            pl.BlockSpec((1, _EMB), lambda i: (0, 0)),
            pl.BlockSpec((1, _EMB), lambda i: (0, 0)),
        ],
        out_specs=pl.BlockSpec((ntile, _EMB), lambda i: (i, 0)),
        out_shape=jax.ShapeDtypeStruct((_N, _EMB), jnp.float32),
    )(
        scale.reshape(1, 1),
        h,
        agg2,
        W1l,
        s1.reshape(1, 2 * _EMB),
        t1.reshape(1, 2 * _EMB),
        W2l,
        s2.reshape(1, _EMB),
        t2.reshape(1, _EMB),
    )


def kernel(x, edge_index, edge_attr, batch, W1, b1, W2, b2, We1, be1, We2, be2,
           bn1_g, bn1_b, bn_g, bn_b, gin_eps):
    src = edge_index[0]
    dst = edge_index[1]
    npad = _EPAD - _E
    ar = jnp.arange(npad, dtype=jnp.int32)
    src_p = jnp.concatenate([src, (ar * 37) % _N]).reshape(_NW, _NCH, _CH)
    dst_p = jnp.concatenate([dst, _N + (ar % 112)]).reshape(_NW, _NCH, _CH)

    inv = 1.0 / jnp.sqrt(1.0 + _BN_EPS)
    h = x
    L = W1.shape[0]
    for l in range(L):
        e = _edge_mlp(edge_attr, We1[l], be1[l], We2[l], be2[l])
        parts = _sc_msgpass(h, e, src_p, dst_p)
        s1 = inv * bn1_g[l]
        t1 = b1[l] * s1 + bn1_b[l]
        s2 = inv * bn_g[l]
        t2 = b2[l] * s2 + bn_b[l]
        h = _node_mlp(h, parts, 1.0 + gin_eps[l], W1[l], s1, t1,
                      W2[l], s2, t2, relu_out=(l < L - 1))
    return h


# speedup vs baseline: 1.1050x; 1.0082x over previous
import functools

import jax
import jax.numpy as jnp
from jax import lax
from jax.experimental import pallas as pl
from jax.experimental.pallas import tpu as pltpu
from jax.experimental.pallas import tpu_sc as plsc

_EMB = 128
_EDIM = 16
_N = 10000
_E = 320000
_BN_EPS = 1e-5

_NW = 32
_CH = 64
_NCH = 159
_EPW = _NCH * _CH
_EPAD = _EPW * _NW
_NAGG = 10112
_RPT = _NAGG // 16
_NBUF = 3
_NI = _NCH // _NBUF


def _edge_mlp(ea, We1l, be1l, We2l, be2l):
    tile = 8192
    grid = (-(-_E // tile),)

    def body(ea_ref, w1_ref, b1_ref, w2_ref, b2_ref, o_ref):
        t = jnp.dot(ea_ref[...].astype(jnp.bfloat16),
                    w1_ref[...].astype(jnp.bfloat16),
                    preferred_element_type=jnp.float32)
        t = jnp.maximum(t + b1_ref[...], 0.0)
        z = (
            jnp.dot(t.astype(jnp.bfloat16), w2_ref[...].astype(jnp.bfloat16),
                    preferred_element_type=jnp.float32) + b2_ref[...]
        )
        lo = jax.lax.bitcast_convert_type(
            z[:, :_EMB // 2].astype(jnp.bfloat16), jnp.uint16
        ).astype(jnp.int32)
        hi = jax.lax.bitcast_convert_type(
            z[:, _EMB // 2:].astype(jnp.bfloat16), jnp.uint16
        ).astype(jnp.int32)
        o_ref[...] = (hi << 16) | lo

    return pl.pallas_call(
        body,
        grid=grid,
        in_specs=[
            pl.BlockSpec((tile, _EDIM), lambda i: (i, 0)),
            pl.BlockSpec((_EDIM, _EMB), lambda i: (0, 0)),
            pl.BlockSpec((1, _EMB), lambda i: (0, 0)),
            pl.BlockSpec((_EMB, _EMB), lambda i: (0, 0)),
            pl.BlockSpec((1, _EMB), lambda i: (0, 0)),
        ],
        out_specs=pl.BlockSpec((tile, _EMB // 2), lambda i: (i, 0)),
        out_shape=jax.ShapeDtypeStruct((_EPAD, _EMB // 2), jnp.int32),
    )(ea, We1l, be1l.reshape(1, _EMB), We2l, be2l.reshape(1, _EMB))


def _sc_msgpass(h, e, src3d, dst3d):
    mesh = plsc.VectorSubcoreMesh(core_axis_name="c", subcore_axis_name="s")

    @functools.partial(
        pl.kernel,
        mesh=mesh,
        compiler_params=pltpu.CompilerParams(needs_layout_passes=False),
        out_type=jax.ShapeDtypeStruct((2, _NAGG, _EMB), jnp.float32),
        scratch_types=[pltpu.VMEM((1, _CH), jnp.int32) for _ in range(2 * _NBUF)]
        + [pltpu.VMEM((_CH, _EMB // 2), jnp.int32) for _ in range(_NBUF)]
        + [pltpu.VMEM((_CH, _EMB), jnp.float32) for _ in range(_NBUF)]
        + [pltpu.VMEM_SHARED((_NAGG, _EMB), jnp.float32)]
        + [pltpu.SemaphoreType.DMA for _ in range(5 * _NBUF)],
    )
    def k(h_hbm, e_hbm, src_hbm, dst_hbm, out_hbm, *rest):
        src_v = rest[0:_NBUF]
        dst_v = rest[_NBUF:2 * _NBUF]
        e_v = rest[2 * _NBUF:3 * _NBUF]
        g_v = rest[3 * _NBUF:4 * _NBUF]
        agg_sh = rest[4 * _NBUF]
        sem_e = rest[4 * _NBUF + 1:4 * _NBUF + 1 + _NBUF]
        sem_g = rest[4 * _NBUF + 1 + _NBUF:4 * _NBUF + 1 + 2 * _NBUF]
        sem_s = rest[4 * _NBUF + 1 + 2 * _NBUF:4 * _NBUF + 1 + 3 * _NBUF]
        sem_si = rest[4 * _NBUF + 1 + 3 * _NBUF:4 * _NBUF + 1 + 4 * _NBUF]
        sem_di = rest[4 * _NBUF + 1 + 4 * _NBUF:]
        c = lax.axis_index("c")
        s = lax.axis_index("s")
        wid = s * 2 + c

        def zrow(r, _):
            for j in range(8):
                g_v[0][r, pl.ds(j * 16, 16)] = jnp.zeros((16,), jnp.float32)
            return 0

        lax.fori_loop(0, _CH, zrow, 0)
        row0 = s * _RPT
        sizes = ((0, 64), (64, 64), (128, 64), (192, 64), (256, 64),
                 (320, 64), (384, 64), (448, 64), (512, 64), (576, 56))
        for off, sz in sizes:
            pltpu.sync_copy(g_v[0].at[pl.ds(0, sz)], agg_sh.at[pl.ds(row0 + off, sz)])
        plsc.subcore_barrier()

        ebase = wid * _EPW

        def src_async(kk, b):
            pltpu.async_copy(src_hbm.at[wid, pl.ds(kk, 1)], src_v[b], sem_si[b])

        def wait_src_idx(b):
            pltpu.make_async_copy(
                src_hbm.at[wid, pl.ds(0, 1)], src_v[b], sem_si[b]
            ).wait()

        def dst_async(kk, b):
            pltpu.async_copy(dst_hbm.at[wid, pl.ds(kk, 1)], dst_v[b], sem_di[b])

        def wait_dst_idx(b):
            pltpu.make_async_copy(
                dst_hbm.at[wid, pl.ds(0, 1)], dst_v[b], sem_di[b]
            ).wait()

        def issue_eg(kk, b):
            pltpu.async_copy(
                e_hbm.at[pl.ds(ebase + kk * _CH, _CH)], e_v[b], sem_e[b]
            )
            pltpu.async_copy(h_hbm.at[src_v[b].at[0]], g_v[b], sem_g[b])

        def wait_in(b):
            pltpu.make_async_copy(
                e_hbm.at[pl.ds(ebase, _CH)], e_v[b], sem_e[b]
            ).wait()
            pltpu.make_async_copy(
                h_hbm.at[src_v[b].at[0]], g_v[b], sem_g[b]
            ).wait()

        def wait_scatter(b):
            pltpu.make_async_copy(
                g_v[b], agg_sh.at[dst_v[b].at[0]], sem_s[b]
            ).wait()

        for j in range(_NBUF):
            pltpu.sync_copy(src_hbm.at[wid, pl.ds(j, 1)], src_v[j])
            pltpu.sync_copy(dst_hbm.at[wid, pl.ds(j, 1)], dst_v[j])
        issue_eg(0, 0)
        issue_eg(1, 1)

        def phase(i, u):
            kk = i * _NBUF + u
            nb = (u + 2) % _NBUF

            def refill():
                wait_scatter(nb)
                dst_async(kk + 2, nb)
                wait_src_idx(nb)
                issue_eg(kk + 2, nb)

            if u == 0:
                @pl.when(i > 0)
                def _():
                    refill()

                @pl.when(i == 0)
                def _():
                    dst_async(2, 2)
                    issue_eg(2, 2)
            else:
                @pl.when(i < _NI - 1)
                def _():
                    refill()

            wait_in(u)

            @pl.when(kk + _NBUF < _NCH)
            def _():
                src_async(kk + _NBUF, u)

            himask = jnp.full((16,), -65536, jnp.int32)

            def ew(rr, _2):
                for rd in range(2):
                    r = rr * 2 + rd
                    for g in range(4):
                        ei = e_v[u][r, pl.ds(16 * g, 16)]
                        elo = plsc.bitcast(ei << 16, jnp.float32)
                        ehi = plsc.bitcast(ei & himask, jnp.float32)
                        dlo = pl.ds(32 * g, 16)
                        dhi = pl.ds(32 * g + 16, 16)
                        glo = g_v[u][r, dlo]
                        ghi = g_v[u][r, dhi]
                        g_v[u][r, dlo] = jnp.maximum(elo + glo, 0.0)
                        g_v[u][r, dhi] = jnp.maximum(ehi + ghi, 0.0)
                return 0

            lax.fori_loop(0, _CH // 2, ew, 0)
            if u < 2:
                @pl.when(i > 0)
                def _():
                    wait_dst_idx(u)
            else:
                wait_dst_idx(u)
            pltpu.async_copy(
                g_v[u], agg_sh.at[dst_v[u].at[0]], sem_s[u], add=True
            )

        def body(i, _):
            for u in range(_NBUF):
                phase(i, u)
            return 0

        lax.fori_loop(0, _NI, body, 0)
        for b in range(_NBUF):
            wait_scatter(b)
        plsc.subcore_barrier()

        for off, sz in sizes:
            pltpu.sync_copy(agg_sh.at[pl.ds(row0 + off, sz)], g_v[0].at[pl.ds(0, sz)])
            pltpu.sync_copy(
                g_v[0].at[pl.ds(0, sz)], out_hbm.at[c, pl.ds(row0 + off, sz)]
            )

    return k(h, e, src3d, dst3d)


def _node_mlp(h, agg2, scale, W1l, s1, t1, W2l, s2, t2, relu_out):
    ntile = 1000
    grid = (_N // ntile,)

    def body(sc_ref, h_ref, a_ref, w1_ref, s1_ref, t1_ref, w2_ref,
             s2_ref, t2_ref, o_ref):
        u = sc_ref[0, 0] * h_ref[...] + a_ref[0] + a_ref[1]
        z = jnp.dot(u.astype(jnp.bfloat16), w1_ref[...].astype(jnp.bfloat16),
                    preferred_element_type=jnp.float32)
        z = jnp.maximum(z * s1_ref[...] + t1_ref[...], 0.0)
        z = jnp.dot(z.astype(jnp.bfloat16), w2_ref[...].astype(jnp.bfloat16),
                    preferred_element_type=jnp.float32)
        z = z * s2_ref[...] + t2_ref[...]
        if relu_out:
            z = jnp.maximum(z, 0.0)
        o_ref[...] = z + h_ref[...]

    return pl.pallas_call(
        body,
        grid=grid,
        in_specs=[
            pl.BlockSpec((1, 1), lambda i: (0, 0), memory_space=pltpu.SMEM),
            pl.BlockSpec((ntile, _EMB), lambda i: (i, 0)),
            pl.BlockSpec((2, ntile, _EMB), lambda i: (0, i, 0)),
            pl.BlockSpec((_EMB, 2 * _EMB), lambda i: (0, 0)),
            pl.BlockSpec((1, 2 * _EMB), lambda i: (0, 0)),
            pl.BlockSpec((1, 2 * _EMB), lambda i: (0, 0)),
            pl.BlockSpec((2 * _EMB, _EMB), lambda i: (0, 0)),
            pl.BlockSpec((1, _EMB), lambda i: (0, 0)),
            pl.BlockSpec((1, _EMB), lambda i: (0, 0)),
        ],
        out_specs=pl.BlockSpec((ntile, _EMB), lambda i: (i, 0)),
        out_shape=jax.ShapeDtypeStruct((_N, _EMB), jnp.float32),
    )(
        scale.reshape(1, 1),
        h,
        agg2,
        W1l,
        s1.reshape(1, 2 * _EMB),
        t1.reshape(1, 2 * _EMB),
        W2l,
        s2.reshape(1, _EMB),
        t2.reshape(1, _EMB),
    )


def kernel(x, edge_index, edge_attr, batch, W1, b1, W2, b2, We1, be1, We2, be2,
           bn1_g, bn1_b, bn_g, bn_b, gin_eps):
    src = edge_index[0]
    dst = edge_index[1]
    npad = _EPAD - _E
    ar = jnp.arange(npad, dtype=jnp.int32)
    src_p = jnp.concatenate([src, (ar * 37) % _N]).reshape(_NW, _NCH, _CH)
    dst_p = jnp.concatenate([dst, _N + (ar % 112)]).reshape(_NW, _NCH, _CH)

    cols = jnp.arange(_EMB, dtype=jnp.int32)
    half, j = cols // 64, cols % 64
    q = 32 * (j // 16) + 16 * half + (j % 16)
    rows = jnp.arange(_EMB, dtype=jnp.int32)
    qmat = (rows[:, None] == q[None, :]).astype(jnp.float32)

    inv = 1.0 / jnp.sqrt(1.0 + _BN_EPS)
    h = x
    L = W1.shape[0]
    for l in range(L):
        e = _edge_mlp(edge_attr, We1[l], be1[l], We2[l] @ qmat,
                      be2[l] @ qmat)
        parts = _sc_msgpass(h, e, src_p, dst_p)
        s1 = inv * bn1_g[l]
        t1 = b1[l] * s1 + bn1_b[l]
        s2 = inv * bn_g[l]
        t2 = b2[l] * s2 + bn_b[l]
        h = _node_mlp(h, parts, 1.0 + gin_eps[l], W1[l], s1, t1,
                      W2[l], s2, t2, relu_out=(l < L - 1))
    return h
